# baseline (speedup 1.0000x reference)
import dataclasses
import functools

import jax
import jax.numpy as jnp
from jax import lax
from jax.experimental import pallas as pl
from jax.experimental.pallas import tpu as pltpu
from jax.experimental.pallas import tpu_sc as plsc

N = 10000
E = 160000
C = 2048
IN = 768
H = 4
D = 64
HID = 128
HD = H * D

NP_ = 10112
ROWS_PT = NP_ // 16
KCH = 128
NB = 10
EP = 16 * NB * 8 * KCH
NCH = EP // KCH

_mesh = plsc.VectorSubcoreMesh(core_axis_name="c", subcore_axis_name="s")

_cp = pltpu.CompilerParams()
if "needs_layout_passes" in pltpu.CompilerParams.__dataclass_fields__:
    _cp = dataclasses.replace(_cp, needs_layout_passes=False)


def _ex_body(tabA, tabB, srcm, dstm,
             exoutA, exoutB,
             tab, srcv, dstv, exbuf0, exbuf1, sem0, sem1):
    c = lax.axis_index("c")
    s = lax.axis_index("s")

    @pl.when(c == 0)
    def _():
        pltpu.sync_copy(tabA, tab)

    @pl.when(c == 1)
    def _():
        pltpu.sync_copy(tabB, tab)

    four = jnp.int32(4)
    iota = jnp.arange(16, dtype=jnp.int32)
    lane_base = (iota & 7) * 16

    @pl.loop(0, NB)
    def _(b):
        pltpu.sync_copy(srcm.at[pl.ds(s * 80 + 8 * b, 8)], srcv)
        pltpu.sync_copy(dstm.at[pl.ds(s * 80 + 8 * b, 8)], dstv)
        handles = [None, None]
        for r in range(8):
            exbuf = exbuf0 if r % 2 == 0 else exbuf1
            sem = sem0 if r % 2 == 0 else sem1
            if handles[r % 2] is not None:
                handles[r % 2].wait()
            for g in range(8):
                sidx = srcv[r, pl.ds(16 * g, 16)] * four
                didx = dstv[r, pl.ds(16 * g, 16)] * four
                el0 = plsc.load_gather(tab, [sidx])
                el1 = plsc.load_gather(tab, [sidx + 1])
                er0 = plsc.load_gather(tab, [didx + 2])
                er1 = plsc.load_gather(tab, [didx + 3])
                rows = (iota >> 3) + (2 * g)
                ev0 = el0 + er0
                ev0 = jnp.maximum(ev0, 0.2 * ev0)
                plsc.store_scatter(exbuf, [rows, lane_base], jnp.exp(ev0))
                ev1 = el1 + er1
                ev1 = jnp.maximum(ev1, 0.2 * ev1)
                plsc.store_scatter(exbuf, [rows, lane_base + 1], jnp.exp(ev1))

            m16 = (s * 80 + 8 * b + r) * 16

            @pl.when(c == 0)
            def _():
                pltpu.async_copy(exbuf, exoutA.at[pl.ds(m16, 16)], sem)

            @pl.when(c == 1)
            def _():
                pltpu.async_copy(exbuf, exoutB.at[pl.ds(m16, 16)], sem)

            handles[r % 2] = pltpu.make_async_copy(
                exbuf, exoutA.at[pl.ds(m16, 16)], sem)
        handles[0].wait()
        handles[1].wait()


_ex_kernel = pl.kernel(
    _ex_body,
    out_type=(
        jax.ShapeDtypeStruct((16 * NCH, 128), jnp.float32),
        jax.ShapeDtypeStruct((16 * NCH, 128), jnp.float32),
    ),
    mesh=_mesh,
    compiler_params=_cp,
    scratch_types=[
        pltpu.VMEM((NP_ * 4,), jnp.float32),
        pltpu.VMEM((8, KCH), jnp.int32),
        pltpu.VMEM((8, KCH), jnp.int32),
        pltpu.VMEM((16, 128), jnp.float32),
        pltpu.VMEM((16, 128), jnp.float32),
        pltpu.SemaphoreType.DMA,
        pltpu.SemaphoreType.DMA,
    ],
)


def _agg_body(featA, featB, srcm, dstm, exoA, exoB,
              accA, accB,
              acc_sh, srcv, dstv, exbuf0, exbuf1, frow, sems, seme0, seme1):
    c = lax.axis_index("c")
    s = lax.axis_index("s")

    zv = jnp.zeros((16,), jnp.float32)

    @pl.loop(0, KCH)
    def _(k):
        for j in range(8):
            frow[k, pl.ds(16 * j, 16)] = zv

    r0 = s * ROWS_PT
    for b in range(4):
        pltpu.sync_copy(frow, acc_sh.at[pl.ds(r0 + 128 * b, 128)])
    pltpu.sync_copy(frow.at[pl.ds(0, 120)], acc_sh.at[pl.ds(r0 + 512, 120)])

    plsc.subcore_barrier()

    @pl.loop(0, NB)
    def _(b):
        pltpu.sync_copy(srcm.at[pl.ds(s * 80 + 8 * b, 8)], srcv)
        pltpu.sync_copy(dstm.at[pl.ds(s * 80 + 8 * b, 8)], dstv)
        for r in range(8):
            m16 = (s * 80 + 8 * b + r) * 16
            exb, seme = (exbuf0, seme0) if r % 2 == 0 else (exbuf1, seme1)
            nexb, nseme = (exbuf1, seme1) if r % 2 == 0 else (exbuf0, seme0)

            if r == 0:
                @pl.when(c == 0)
                def _():
                    pltpu.sync_copy(exoA.at[pl.ds(m16, 16)], exb)

                @pl.when(c == 1)
                def _():
                    pltpu.sync_copy(exoB.at[pl.ds(m16, 16)], exb)
            else:
                pltpu.make_async_copy(frow, acc_sh.at[dstv.at[r - 1]], sems).wait()

            if r < 7:
                @pl.when(c == 0)
                def _():
                    pltpu.async_copy(exoA.at[pl.ds(m16 + 16, 16)], nexb, nseme)

                @pl.when(c == 1)
                def _():
                    pltpu.async_copy(exoB.at[pl.ds(m16 + 16, 16)], nexb, nseme)

            @pl.when(c == 0)
            def _():
                pltpu.sync_copy(featA.at[srcv.at[r]], frow)

            @pl.when(c == 1)
            def _():
                pltpu.sync_copy(featB.at[srcv.at[r]], frow)

            if r > 0:
                pltpu.make_async_copy(exoA.at[pl.ds(m16, 16)], exb, seme).wait()

            @plsc.parallel_loop(0, KCH, unroll=8)
            def _(k):
                pair = exb[k >> 3, pl.ds((k & 7) * 16, 16)]
                s0 = pair[0]
                s1 = pair[1]
                for j in range(4):
                    frow[k, pl.ds(16 * j, 16)] = frow[k, pl.ds(16 * j, 16)] * s0
                for j in range(4, 8):
                    frow[k, pl.ds(16 * j, 16)] = frow[k, pl.ds(16 * j, 16)] * s1

            pltpu.async_copy(frow, acc_sh.at[dstv.at[r]], sems, add=True)

        pltpu.make_async_copy(frow, acc_sh.at[dstv.at[7]], sems).wait()

    plsc.subcore_barrier()

    def _wout(acc_out):
        for b in range(4):
            pltpu.sync_copy(acc_sh.at[pl.ds(r0 + 128 * b, 128)], frow)
            pltpu.sync_copy(frow, acc_out.at[pl.ds(r0 + 128 * b, 128)])
        pltpu.sync_copy(acc_sh.at[pl.ds(r0 + 512, 120)], frow.at[pl.ds(0, 120)])
        pltpu.sync_copy(frow.at[pl.ds(0, 120)], acc_out.at[pl.ds(r0 + 512, 120)])

    @pl.when(c == 0)
    def _():
        _wout(accA)

    @pl.when(c == 1)
    def _():
        _wout(accB)


_agg_kernel = pl.kernel(
    _agg_body,
    out_type=(
        jax.ShapeDtypeStruct((NP_, 128), jnp.float32),
        jax.ShapeDtypeStruct((NP_, 128), jnp.float32),
    ),
    mesh=_mesh,
    compiler_params=_cp,
    scratch_types=[
        pltpu.VMEM_SHARED((NP_, 128), jnp.float32),
        pltpu.VMEM((8, KCH), jnp.int32),
        pltpu.VMEM((8, KCH), jnp.int32),
        pltpu.VMEM((16, 128), jnp.float32),
        pltpu.VMEM((16, 128), jnp.float32),
        pltpu.VMEM((KCH, 128), jnp.float32),
        pltpu.SemaphoreType.DMA,
        pltpu.SemaphoreType.DMA,
        pltpu.SemaphoreType.DMA,
    ],
)


def _den_body(dstm, exoA, exoB,
              denA, denB,
              den_sh, dstv, exbuf0, exbuf1, exwide, sems, seme0, seme1):
    c = lax.axis_index("c")
    s = lax.axis_index("s")

    zv = jnp.zeros((16,), jnp.float32)

    @pl.loop(0, KCH)
    def _(k):
        for j in range(8):
            exwide[k, pl.ds(16 * j, 16)] = zv

    r0 = s * ROWS_PT
    for b in range(4):
        pltpu.sync_copy(exwide, den_sh.at[pl.ds(r0 + 128 * b, 128)])
    pltpu.sync_copy(exwide.at[pl.ds(0, 120)], den_sh.at[pl.ds(r0 + 512, 120)])

    plsc.subcore_barrier()

    @pl.loop(0, NB)
    def _(b):
        pltpu.sync_copy(dstm.at[pl.ds(s * 80 + 8 * b, 8)], dstv)
        for r in range(8):
            m16 = (s * 80 + 8 * b + r) * 16
            exb, seme = (exbuf0, seme0) if r % 2 == 0 else (exbuf1, seme1)
            nexb, nseme = (exbuf1, seme1) if r % 2 == 0 else (exbuf0, seme0)

            if r == 0:
                @pl.when(c == 0)
                def _():
                    pltpu.sync_copy(exoA.at[pl.ds(m16, 16)], exb)

                @pl.when(c == 1)
                def _():
                    pltpu.sync_copy(exoB.at[pl.ds(m16, 16)], exb)
            else:
                pltpu.make_async_copy(exwide, den_sh.at[dstv.at[r - 1]], sems).wait()

            if r < 7:
                @pl.when(c == 0)
                def _():
                    pltpu.async_copy(exoA.at[pl.ds(m16 + 16, 16)], nexb, nseme)

                @pl.when(c == 1)
                def _():
                    pltpu.async_copy(exoB.at[pl.ds(m16 + 16, 16)], nexb, nseme)

            if r > 0:
                pltpu.make_async_copy(exoA.at[pl.ds(m16, 16)], exb, seme).wait()

            @plsc.parallel_loop(0, KCH, unroll=8)
            def _(k):
                pair = exb[k >> 3, pl.ds((k & 7) * 16, 16)]
                v0 = pair * jnp.float32(0) + pair[0]
                v1 = pair * jnp.float32(0) + pair[1]
                for j in range(4):
                    exwide[k, pl.ds(16 * j, 16)] = v0
                for j in range(4, 8):
                    exwide[k, pl.ds(16 * j, 16)] = v1

            pltpu.async_copy(exwide, den_sh.at[dstv.at[r]], sems, add=True)

        pltpu.make_async_copy(exwide, den_sh.at[dstv.at[7]], sems).wait()

    plsc.subcore_barrier()

    def _wout(den_out):
        for b in range(4):
            pltpu.sync_copy(den_sh.at[pl.ds(r0 + 128 * b, 128)], exwide)
            pltpu.sync_copy(exwide, den_out.at[pl.ds(r0 + 128 * b, 128)])
        pltpu.sync_copy(den_sh.at[pl.ds(r0 + 512, 120)], exwide.at[pl.ds(0, 120)])
        pltpu.sync_copy(exwide.at[pl.ds(0, 120)], den_out.at[pl.ds(r0 + 512, 120)])

    @pl.when(c == 0)
    def _():
        _wout(denA)

    @pl.when(c == 1)
    def _():
        _wout(denB)


_den_kernel = pl.kernel(
    _den_body,
    out_type=(
        jax.ShapeDtypeStruct((NP_, 128), jnp.float32),
        jax.ShapeDtypeStruct((NP_, 128), jnp.float32),
    ),
    mesh=_mesh,
    compiler_params=_cp,
    scratch_types=[
        pltpu.VMEM_SHARED((NP_, 128), jnp.float32),
        pltpu.VMEM((8, KCH), jnp.int32),
        pltpu.VMEM((16, 128), jnp.float32),
        pltpu.VMEM((16, 128), jnp.float32),
        pltpu.VMEM((KCH, 128), jnp.float32),
        pltpu.SemaphoreType.DMA,
        pltpu.SemaphoreType.DMA,
        pltpu.SemaphoreType.DMA,
    ],
)


def _gat(x, edge_index, W, al, ar, bias):
    feat = x @ W
    fr = feat.reshape(N, H, D)
    el = (fr * al[None]).sum(-1)
    er = (fr * ar[None]).sum(-1)

    featA = jnp.zeros((NP_, 128), jnp.float32).at[:N].set(feat[:, :128])
    featB = jnp.zeros((NP_, 128), jnp.float32).at[:N].set(feat[:, 128:])

    def tab_for(h0):
        t = jnp.zeros((NP_, 4), jnp.float32)
        t = t.at[:N, 0].set(el[:, h0]).at[:N, 1].set(el[:, h0 + 1])
        t = t.at[:N, 2].set(er[:, h0]).at[:N, 3].set(er[:, h0 + 1])
        return t.reshape(-1)

    tabA = tab_for(0)
    tabB = tab_for(2)
    srcm = jnp.full((EP,), N, jnp.int32).at[:E].set(edge_index[0]).reshape(NCH, KCH)
    dstm = jnp.full((EP,), N, jnp.int32).at[:E].set(edge_index[1]).reshape(NCH, KCH)

    exoA, exoB = _ex_kernel(tabA, tabB, srcm, dstm)
    accA, accB = _agg_kernel(featA, featB, srcm, dstm, exoA, exoB)
    denA, denB = _den_kernel(dstm, exoA, exoB)

    acc = jnp.concatenate([accA[:N], accB[:N]], axis=1).reshape(N, H, D)
    den4 = jnp.concatenate(
        [denA[:N, 0:1], denA[:N, 64:65], denB[:N, 0:1], denB[:N, 64:65]], axis=1)
    den4 = jnp.maximum(den4, 1e-16)
    rst = acc / den4[:, :, None] + bias.reshape(1, H, D)
    return jax.nn.elu(rst)


def _sem_att(z, W1, b1, W2):
    h = jnp.tanh(z @ W1 + b1)
    w = (h @ W2).mean(0)
    beta = jax.nn.softmax(w, axis=0)
    return (beta[None, :, :] * z).sum(1)


def _overwrite_mean(h_nhd, char, semm):
    base = h_nhd.mean(axis=1)
    idx = jnp.arange(C)
    eq = char[None, :] == char[:, None]
    has_later = (eq & (idx[None, :] > idx[:, None])).any(axis=1)
    scat_idx = jnp.where(has_later, C + N, char)
    sel = jnp.zeros((N,), jnp.bool_).at[char].set(True)
    base = base * (~sel)[:, None]
    delta = jnp.zeros((N, D), jnp.float32).at[scat_idx].add(semm, mode="drop")
    return base + delta


def kernel(x0, x1, x2, edge_index0, edge_index1, edge_index2, char0, char1, char2, W0, al0, ar0, bias0, W1, al1, ar1, bias1, W2, al2, ar2, bias2, sW1, sb1, sW2, aW1, ab1, aW2):
    h1 = _gat(x0, edge_index0, W0, al0, ar0, bias0)
    hh = _gat(x2, edge_index2, W2, al2, ar2, bias2)
    h2 = _gat(x1, edge_index1, W1, al1, ar1, bias1)

    se = jnp.stack([h1[char0].reshape(C, HD), hh[char2].reshape(C, HD)], axis=1)
    s = _sem_att(se, sW1, sb1, sW2)
    se2 = jnp.stack([s, h2[char1].reshape(C, HD)], axis=1)
    sem = _sem_att(se2, aW1, ab1, aW2)
    semr = sem.reshape(C, H, D)
    semm = semr.mean(axis=1)

    o1 = _overwrite_mean(h1, char0, semm)
    o2 = _overwrite_mean(h2, char1, semm)
    o3 = _overwrite_mean(hh, char2, semm)
    return (semr, o1, o2, o3)


# speedup vs baseline: 1.0317x; 1.0317x over previous
import dataclasses
import functools

import jax
import jax.numpy as jnp
from jax import lax
from jax.experimental import pallas as pl
from jax.experimental.pallas import tpu as pltpu
from jax.experimental.pallas import tpu_sc as plsc

N = 10000
E = 160000
C = 2048
IN = 768
H = 4
D = 64
HID = 128
HD = H * D

NP_ = 10112
ROWS_PT = NP_ // 16
KCH = 128
NB = 10
EP = 16 * NB * 8 * KCH
NCH = EP // KCH

_mesh = plsc.VectorSubcoreMesh(core_axis_name="c", subcore_axis_name="s")

_cp = pltpu.CompilerParams()
if "needs_layout_passes" in pltpu.CompilerParams.__dataclass_fields__:
    _cp = dataclasses.replace(_cp, needs_layout_passes=False)


def _ex_body(tabA, tabB, srcm, dstm,
             exoutA, exoutB,
             tab, srcv, dstv, exbuf0, exbuf1, sem0, sem1):
    c = lax.axis_index("c")
    s = lax.axis_index("s")

    @pl.when(c == 0)
    def _():
        pltpu.sync_copy(tabA, tab)

    @pl.when(c == 1)
    def _():
        pltpu.sync_copy(tabB, tab)

    four = jnp.int32(4)
    iota = jnp.arange(16, dtype=jnp.int32)
    lane_base = (iota & 7) * 16

    @pl.loop(0, NB)
    def _(b):
        pltpu.sync_copy(srcm.at[pl.ds(s * 80 + 8 * b, 8)], srcv)
        pltpu.sync_copy(dstm.at[pl.ds(s * 80 + 8 * b, 8)], dstv)
        handles = [None, None]
        for r in range(8):
            exbuf = exbuf0 if r % 2 == 0 else exbuf1
            sem = sem0 if r % 2 == 0 else sem1
            if handles[r % 2] is not None:
                handles[r % 2].wait()
            for g in range(8):
                sidx = srcv[r, pl.ds(16 * g, 16)] * four
                didx = dstv[r, pl.ds(16 * g, 16)] * four
                el0 = plsc.load_gather(tab, [sidx])
                el1 = plsc.load_gather(tab, [sidx + 1])
                er0 = plsc.load_gather(tab, [didx + 2])
                er1 = plsc.load_gather(tab, [didx + 3])
                rows = (iota >> 3) + (2 * g)
                ev0 = el0 + er0
                ev0 = jnp.maximum(ev0, 0.2 * ev0)
                plsc.store_scatter(exbuf, [rows, lane_base], jnp.exp(ev0))
                ev1 = el1 + er1
                ev1 = jnp.maximum(ev1, 0.2 * ev1)
                plsc.store_scatter(exbuf, [rows, lane_base + 1], jnp.exp(ev1))

            m16 = (s * 80 + 8 * b + r) * 16

            @pl.when(c == 0)
            def _():
                pltpu.async_copy(exbuf, exoutA.at[pl.ds(m16, 16)], sem)

            @pl.when(c == 1)
            def _():
                pltpu.async_copy(exbuf, exoutB.at[pl.ds(m16, 16)], sem)

            handles[r % 2] = pltpu.make_async_copy(
                exbuf, exoutA.at[pl.ds(m16, 16)], sem)
        handles[0].wait()
        handles[1].wait()


_ex_kernel = pl.kernel(
    _ex_body,
    out_type=(
        jax.ShapeDtypeStruct((16 * NCH, 128), jnp.float32),
        jax.ShapeDtypeStruct((16 * NCH, 128), jnp.float32),
    ),
    mesh=_mesh,
    compiler_params=_cp,
    scratch_types=[
        pltpu.VMEM((NP_ * 4,), jnp.float32),
        pltpu.VMEM((8, KCH), jnp.int32),
        pltpu.VMEM((8, KCH), jnp.int32),
        pltpu.VMEM((16, 128), jnp.float32),
        pltpu.VMEM((16, 128), jnp.float32),
        pltpu.SemaphoreType.DMA,
        pltpu.SemaphoreType.DMA,
    ],
)


def _agg_body(featA, featB, srcm, dstm, exoA, exoB,
              accA, accB,
              acc_sh, srcv, dstv, exbuf0, exbuf1, frow, sems, seme0, seme1):
    c = lax.axis_index("c")
    s = lax.axis_index("s")

    zv = jnp.zeros((16,), jnp.float32)

    @pl.loop(0, KCH)
    def _(k):
        for j in range(8):
            frow[k, pl.ds(16 * j, 16)] = zv

    r0 = s * ROWS_PT
    for b in range(4):
        pltpu.sync_copy(frow, acc_sh.at[pl.ds(r0 + 128 * b, 128)])
    pltpu.sync_copy(frow.at[pl.ds(0, 120)], acc_sh.at[pl.ds(r0 + 512, 120)])

    plsc.subcore_barrier()

    @pl.loop(0, NB)
    def _(b):
        pltpu.sync_copy(srcm.at[pl.ds(s * 80 + 8 * b, 8)], srcv)
        pltpu.sync_copy(dstm.at[pl.ds(s * 80 + 8 * b, 8)], dstv)
        for r in range(8):
            m16 = (s * 80 + 8 * b + r) * 16
            exb, seme = (exbuf0, seme0) if r % 2 == 0 else (exbuf1, seme1)
            nexb, nseme = (exbuf1, seme1) if r % 2 == 0 else (exbuf0, seme0)

            if r == 0:
                @pl.when(c == 0)
                def _():
                    pltpu.sync_copy(exoA.at[pl.ds(m16, 16)], exb)

                @pl.when(c == 1)
                def _():
                    pltpu.sync_copy(exoB.at[pl.ds(m16, 16)], exb)
            else:
                pltpu.make_async_copy(frow, acc_sh.at[dstv.at[r - 1]], sems).wait()

            if r < 7:
                @pl.when(c == 0)
                def _():
                    pltpu.async_copy(exoA.at[pl.ds(m16 + 16, 16)], nexb, nseme)

                @pl.when(c == 1)
                def _():
                    pltpu.async_copy(exoB.at[pl.ds(m16 + 16, 16)], nexb, nseme)

            @pl.when(c == 0)
            def _():
                pltpu.sync_copy(featA.at[srcv.at[r]], frow)

            @pl.when(c == 1)
            def _():
                pltpu.sync_copy(featB.at[srcv.at[r]], frow)

            if r > 0:
                pltpu.make_async_copy(exoA.at[pl.ds(m16, 16)], exb, seme).wait()

            @plsc.parallel_loop(0, KCH, unroll=4)
            def _(k):
                pair = exb[k >> 3, pl.ds((k & 7) * 16, 16)]
                s0 = pair[0]
                s1 = pair[1]
                for j in range(4):
                    frow[k, pl.ds(16 * j, 16)] = frow[k, pl.ds(16 * j, 16)] * s0
                for j in range(4, 8):
                    frow[k, pl.ds(16 * j, 16)] = frow[k, pl.ds(16 * j, 16)] * s1

            pltpu.async_copy(frow, acc_sh.at[dstv.at[r]], sems, add=True)

        pltpu.make_async_copy(frow, acc_sh.at[dstv.at[7]], sems).wait()

    plsc.subcore_barrier()

    def _wout(acc_out):
        for b in range(4):
            pltpu.sync_copy(acc_sh.at[pl.ds(r0 + 128 * b, 128)], frow)
            pltpu.sync_copy(frow, acc_out.at[pl.ds(r0 + 128 * b, 128)])
        pltpu.sync_copy(acc_sh.at[pl.ds(r0 + 512, 120)], frow.at[pl.ds(0, 120)])
        pltpu.sync_copy(frow.at[pl.ds(0, 120)], acc_out.at[pl.ds(r0 + 512, 120)])

    @pl.when(c == 0)
    def _():
        _wout(accA)

    @pl.when(c == 1)
    def _():
        _wout(accB)


_agg_kernel = pl.kernel(
    _agg_body,
    out_type=(
        jax.ShapeDtypeStruct((NP_, 128), jnp.float32),
        jax.ShapeDtypeStruct((NP_, 128), jnp.float32),
    ),
    mesh=_mesh,
    compiler_params=_cp,
    scratch_types=[
        pltpu.VMEM_SHARED((NP_, 128), jnp.float32),
        pltpu.VMEM((8, KCH), jnp.int32),
        pltpu.VMEM((8, KCH), jnp.int32),
        pltpu.VMEM((16, 128), jnp.float32),
        pltpu.VMEM((16, 128), jnp.float32),
        pltpu.VMEM((KCH, 128), jnp.float32),
        pltpu.SemaphoreType.DMA,
        pltpu.SemaphoreType.DMA,
        pltpu.SemaphoreType.DMA,
    ],
)


def _den_body(dstm, exoA, exoB,
              denA, denB,
              den_sh, dstv, exbuf0, exbuf1, exwide, sems, seme0, seme1):
    c = lax.axis_index("c")
    s = lax.axis_index("s")

    zv = jnp.zeros((16,), jnp.float32)

    @pl.loop(0, KCH)
    def _(k):
        for j in range(8):
            exwide[k, pl.ds(16 * j, 16)] = zv

    r0 = s * ROWS_PT
    for b in range(4):
        pltpu.sync_copy(exwide, den_sh.at[pl.ds(r0 + 128 * b, 128)])
    pltpu.sync_copy(exwide.at[pl.ds(0, 120)], den_sh.at[pl.ds(r0 + 512, 120)])

    plsc.subcore_barrier()

    @pl.loop(0, NB)
    def _(b):
        pltpu.sync_copy(dstm.at[pl.ds(s * 80 + 8 * b, 8)], dstv)
        for r in range(8):
            m16 = (s * 80 + 8 * b + r) * 16
            exb, seme = (exbuf0, seme0) if r % 2 == 0 else (exbuf1, seme1)
            nexb, nseme = (exbuf1, seme1) if r % 2 == 0 else (exbuf0, seme0)

            if r == 0:
                @pl.when(c == 0)
                def _():
                    pltpu.sync_copy(exoA.at[pl.ds(m16, 16)], exb)

                @pl.when(c == 1)
                def _():
                    pltpu.sync_copy(exoB.at[pl.ds(m16, 16)], exb)
            else:
                pltpu.make_async_copy(exwide, den_sh.at[dstv.at[r - 1]], sems).wait()

            if r < 7:
                @pl.when(c == 0)
                def _():
                    pltpu.async_copy(exoA.at[pl.ds(m16 + 16, 16)], nexb, nseme)

                @pl.when(c == 1)
                def _():
                    pltpu.async_copy(exoB.at[pl.ds(m16 + 16, 16)], nexb, nseme)

            if r > 0:
                pltpu.make_async_copy(exoA.at[pl.ds(m16, 16)], exb, seme).wait()

            @plsc.parallel_loop(0, KCH, unroll=4)
            def _(k):
                pair = exb[k >> 3, pl.ds((k & 7) * 16, 16)]
                v0 = pair * jnp.float32(0) + pair[0]
                v1 = pair * jnp.float32(0) + pair[1]
                for j in range(4):
                    exwide[k, pl.ds(16 * j, 16)] = v0
                for j in range(4, 8):
                    exwide[k, pl.ds(16 * j, 16)] = v1

            pltpu.async_copy(exwide, den_sh.at[dstv.at[r]], sems, add=True)

        pltpu.make_async_copy(exwide, den_sh.at[dstv.at[7]], sems).wait()

    plsc.subcore_barrier()

    def _wout(den_out):
        for b in range(4):
            pltpu.sync_copy(den_sh.at[pl.ds(r0 + 128 * b, 128)], exwide)
            pltpu.sync_copy(exwide, den_out.at[pl.ds(r0 + 128 * b, 128)])
        pltpu.sync_copy(den_sh.at[pl.ds(r0 + 512, 120)], exwide.at[pl.ds(0, 120)])
        pltpu.sync_copy(exwide.at[pl.ds(0, 120)], den_out.at[pl.ds(r0 + 512, 120)])

    @pl.when(c == 0)
    def _():
        _wout(denA)

    @pl.when(c == 1)
    def _():
        _wout(denB)


_den_kernel = pl.kernel(
    _den_body,
    out_type=(
        jax.ShapeDtypeStruct((NP_, 128), jnp.float32),
        jax.ShapeDtypeStruct((NP_, 128), jnp.float32),
    ),
    mesh=_mesh,
    compiler_params=_cp,
    scratch_types=[
        pltpu.VMEM_SHARED((NP_, 128), jnp.float32),
        pltpu.VMEM((8, KCH), jnp.int32),
        pltpu.VMEM((16, 128), jnp.float32),
        pltpu.VMEM((16, 128), jnp.float32),
        pltpu.VMEM((KCH, 128), jnp.float32),
        pltpu.SemaphoreType.DMA,
        pltpu.SemaphoreType.DMA,
        pltpu.SemaphoreType.DMA,
    ],
)


def _feat_body(x_ref, W_ref, twA_ref, twB_ref, fA_ref, fB_ref, tA_ref, tB_ref):
    f = jnp.dot(x_ref[...], W_ref[...], preferred_element_type=jnp.float32)
    fA_ref[...] = f[:, :128]
    fB_ref[...] = f[:, 128:]
    tA_ref[...] = jnp.dot(f, twA_ref[...], preferred_element_type=jnp.float32)
    tB_ref[...] = jnp.dot(f, twB_ref[...], preferred_element_type=jnp.float32)


_BN = 1264

_feat_kernel = pl.pallas_call(
    _feat_body,
    grid=(NP_ // _BN,),
    in_specs=[
        pl.BlockSpec((_BN, IN), lambda i: (i, 0)),
        pl.BlockSpec((IN, HD), lambda i: (0, 0)),
        pl.BlockSpec((HD, 8), lambda i: (0, 0)),
        pl.BlockSpec((HD, 8), lambda i: (0, 0)),
    ],
    out_specs=[
        pl.BlockSpec((_BN, 128), lambda i: (i, 0)),
        pl.BlockSpec((_BN, 128), lambda i: (i, 0)),
        pl.BlockSpec((_BN, 8), lambda i: (i, 0)),
        pl.BlockSpec((_BN, 8), lambda i: (i, 0)),
    ],
    out_shape=(
        jax.ShapeDtypeStruct((NP_, 128), jnp.float32),
        jax.ShapeDtypeStruct((NP_, 128), jnp.float32),
        jax.ShapeDtypeStruct((NP_, 8), jnp.float32),
        jax.ShapeDtypeStruct((NP_, 8), jnp.float32),
    ),
)


def _gat(x, edge_index, W, al, ar, bias):
    def tw_for(h0):
        t = jnp.zeros((HD, 8), jnp.float32)
        for j, (vec, h) in enumerate([(al, h0), (al, h0 + 1), (ar, h0), (ar, h0 + 1)]):
            t = t.at[h * D:(h + 1) * D, j].set(vec[h])
        return t

    x_pad = jnp.zeros((NP_, IN), jnp.float32).at[:N].set(x)
    featA, featB, tabA8, tabB8 = _feat_kernel(x_pad, W, tw_for(0), tw_for(2))
    tabA = tabA8[:, :4].reshape(-1)
    tabB = tabB8[:, :4].reshape(-1)
    srcm = jnp.full((EP,), N, jnp.int32).at[:E].set(edge_index[0]).reshape(NCH, KCH)
    dstm = jnp.full((EP,), N, jnp.int32).at[:E].set(edge_index[1]).reshape(NCH, KCH)

    exoA, exoB = _ex_kernel(tabA, tabB, srcm, dstm)
    accA, accB = _agg_kernel(featA, featB, srcm, dstm, exoA, exoB)
    denA, denB = _den_kernel(dstm, exoA, exoB)

    acc = jnp.concatenate([accA[:N], accB[:N]], axis=1).reshape(N, H, D)
    den4 = jnp.concatenate(
        [denA[:N, 0:1], denA[:N, 64:65], denB[:N, 0:1], denB[:N, 64:65]], axis=1)
    den4 = jnp.maximum(den4, 1e-16)
    rst = acc / den4[:, :, None] + bias.reshape(1, H, D)
    return jax.nn.elu(rst)


def _sem_att(z, W1, b1, W2):
    h = jnp.tanh(z @ W1 + b1)
    w = (h @ W2).mean(0)
    beta = jax.nn.softmax(w, axis=0)
    return (beta[None, :, :] * z).sum(1)


def _overwrite_mean(h_nhd, char, semm):
    base = h_nhd.mean(axis=1)
    idx = jnp.arange(C)
    eq = char[None, :] == char[:, None]
    has_later = (eq & (idx[None, :] > idx[:, None])).any(axis=1)
    scat_idx = jnp.where(has_later, C + N, char)
    sel = jnp.zeros((N,), jnp.bool_).at[char].set(True)
    base = base * (~sel)[:, None]
    delta = jnp.zeros((N, D), jnp.float32).at[scat_idx].add(semm, mode="drop")
    return base + delta


def kernel(x0, x1, x2, edge_index0, edge_index1, edge_index2, char0, char1, char2, W0, al0, ar0, bias0, W1, al1, ar1, bias1, W2, al2, ar2, bias2, sW1, sb1, sW2, aW1, ab1, aW2):
    h1 = _gat(x0, edge_index0, W0, al0, ar0, bias0)
    hh = _gat(x2, edge_index2, W2, al2, ar2, bias2)
    h2 = _gat(x1, edge_index1, W1, al1, ar1, bias1)

    se = jnp.stack([h1[char0].reshape(C, HD), hh[char2].reshape(C, HD)], axis=1)
    s = _sem_att(se, sW1, sb1, sW2)
    se2 = jnp.stack([s, h2[char1].reshape(C, HD)], axis=1)
    sem = _sem_att(se2, aW1, ab1, aW2)
    semr = sem.reshape(C, H, D)
    semm = semr.mean(axis=1)

    o1 = _overwrite_mean(h1, char0, semm)
    o2 = _overwrite_mean(h2, char1, semm)
    o3 = _overwrite_mean(hh, char2, semm)
    return (semr, o1, o2, o3)


# speedup vs baseline: 1.0724x; 1.0394x over previous
import dataclasses
import functools

import jax
import jax.numpy as jnp
from jax import lax
from jax.experimental import pallas as pl
from jax.experimental.pallas import tpu as pltpu
from jax.experimental.pallas import tpu_sc as plsc

N = 10000
E = 160000
C = 2048
IN = 768
H = 4
D = 64
HID = 128
HD = H * D

NP_ = 10112
ROWS_PT = NP_ // 16
KCH = 128
NB = 10
EP = 16 * NB * 8 * KCH
NCH = EP // KCH

_mesh = plsc.VectorSubcoreMesh(core_axis_name="c", subcore_axis_name="s")

_cp = pltpu.CompilerParams()
if "needs_layout_passes" in pltpu.CompilerParams.__dataclass_fields__:
    _cp = dataclasses.replace(_cp, needs_layout_passes=False)


def _ex_body(tabA, tabB, srcm, dstm,
             exoutA, exoutB,
             tab, srcv, dstv, exbuf0, exbuf1, sem0, sem1):
    c = lax.axis_index("c")
    s = lax.axis_index("s")

    @pl.when(c == 0)
    def _():
        pltpu.sync_copy(tabA, tab)

    @pl.when(c == 1)
    def _():
        pltpu.sync_copy(tabB, tab)

    four = jnp.int32(4)
    iota = jnp.arange(16, dtype=jnp.int32)
    lane_base = (iota & 7) * 16

    @pl.loop(0, NB)
    def _(b):
        pltpu.sync_copy(srcm.at[pl.ds(s * 80 + 8 * b, 8)], srcv)
        pltpu.sync_copy(dstm.at[pl.ds(s * 80 + 8 * b, 8)], dstv)
        handles = [None, None]
        for r in range(8):
            exbuf = exbuf0 if r % 2 == 0 else exbuf1
            sem = sem0 if r % 2 == 0 else sem1
            if handles[r % 2] is not None:
                handles[r % 2].wait()
            for g in range(8):
                sidx = srcv[r, pl.ds(16 * g, 16)] * four
                didx = dstv[r, pl.ds(16 * g, 16)] * four
                el0 = plsc.load_gather(tab, [sidx])
                el1 = plsc.load_gather(tab, [sidx + 1])
                er0 = plsc.load_gather(tab, [didx + 2])
                er1 = plsc.load_gather(tab, [didx + 3])
                rows = (iota >> 3) + (2 * g)
                ev0 = el0 + er0
                ev0 = jnp.maximum(ev0, 0.2 * ev0)
                plsc.store_scatter(exbuf, [rows, lane_base], jnp.exp(ev0))
                ev1 = el1 + er1
                ev1 = jnp.maximum(ev1, 0.2 * ev1)
                plsc.store_scatter(exbuf, [rows, lane_base + 1], jnp.exp(ev1))

            m16 = (s * 80 + 8 * b + r) * 16

            @pl.when(c == 0)
            def _():
                pltpu.async_copy(exbuf, exoutA.at[pl.ds(m16, 16)], sem)

            @pl.when(c == 1)
            def _():
                pltpu.async_copy(exbuf, exoutB.at[pl.ds(m16, 16)], sem)

            handles[r % 2] = pltpu.make_async_copy(
                exbuf, exoutA.at[pl.ds(m16, 16)], sem)
        handles[0].wait()
        handles[1].wait()


_ex_kernel = pl.kernel(
    _ex_body,
    out_type=(
        jax.ShapeDtypeStruct((16 * NCH, 128), jnp.float32),
        jax.ShapeDtypeStruct((16 * NCH, 128), jnp.float32),
    ),
    mesh=_mesh,
    compiler_params=_cp,
    scratch_types=[
        pltpu.VMEM((NP_ * 4,), jnp.float32),
        pltpu.VMEM((8, KCH), jnp.int32),
        pltpu.VMEM((8, KCH), jnp.int32),
        pltpu.VMEM((16, 128), jnp.float32),
        pltpu.VMEM((16, 128), jnp.float32),
        pltpu.SemaphoreType.DMA,
        pltpu.SemaphoreType.DMA,
    ],
)


def _agg_body(featA, featB, srcm, dstm, exoA, exoB,
              accA, accB,
              acc_sh, srcv, dstv, exbuf0, exbuf1, frow, sems, seme0, seme1):
    c = lax.axis_index("c")
    s = lax.axis_index("s")

    zv = jnp.zeros((16,), jnp.float32)

    @pl.loop(0, KCH)
    def _(k):
        for j in range(8):
            frow[k, pl.ds(16 * j, 16)] = zv

    r0 = s * ROWS_PT
    for b in range(4):
        pltpu.sync_copy(frow, acc_sh.at[pl.ds(r0 + 128 * b, 128)])
    pltpu.sync_copy(frow.at[pl.ds(0, 120)], acc_sh.at[pl.ds(r0 + 512, 120)])

    plsc.subcore_barrier()

    @pl.loop(0, NB)
    def _(b):
        pltpu.sync_copy(srcm.at[pl.ds(s * 80 + 8 * b, 8)], srcv)
        pltpu.sync_copy(dstm.at[pl.ds(s * 80 + 8 * b, 8)], dstv)
        for r in range(8):
            m16 = (s * 80 + 8 * b + r) * 16
            exb, seme = (exbuf0, seme0) if r % 2 == 0 else (exbuf1, seme1)
            nexb, nseme = (exbuf1, seme1) if r % 2 == 0 else (exbuf0, seme0)

            if r == 0:
                @pl.when(c == 0)
                def _():
                    pltpu.sync_copy(exoA.at[pl.ds(m16, 16)], exb)

                @pl.when(c == 1)
                def _():
                    pltpu.sync_copy(exoB.at[pl.ds(m16, 16)], exb)
            else:
                pltpu.make_async_copy(frow, acc_sh.at[dstv.at[r - 1]], sems).wait()

            if r < 7:
                @pl.when(c == 0)
                def _():
                    pltpu.async_copy(exoA.at[pl.ds(m16 + 16, 16)], nexb, nseme)

                @pl.when(c == 1)
                def _():
                    pltpu.async_copy(exoB.at[pl.ds(m16 + 16, 16)], nexb, nseme)

            @pl.when(c == 0)
            def _():
                pltpu.sync_copy(featA.at[srcv.at[r]], frow)

            @pl.when(c == 1)
            def _():
                pltpu.sync_copy(featB.at[srcv.at[r]], frow)

            if r > 0:
                pltpu.make_async_copy(exoA.at[pl.ds(m16, 16)], exb, seme).wait()

            @plsc.parallel_loop(0, KCH, unroll=4)
            def _(k):
                pair = exb[k >> 3, pl.ds((k & 7) * 16, 16)]
                s0 = pair[0]
                s1 = pair[1]
                for j in range(4):
                    frow[k, pl.ds(16 * j, 16)] = frow[k, pl.ds(16 * j, 16)] * s0
                for j in range(4, 8):
                    frow[k, pl.ds(16 * j, 16)] = frow[k, pl.ds(16 * j, 16)] * s1

            pltpu.async_copy(frow, acc_sh.at[dstv.at[r]], sems, add=True)

        pltpu.make_async_copy(frow, acc_sh.at[dstv.at[7]], sems).wait()

    plsc.subcore_barrier()

    def _wout(acc_out):
        for b in range(4):
            pltpu.sync_copy(acc_sh.at[pl.ds(r0 + 128 * b, 128)], frow)
            pltpu.sync_copy(frow, acc_out.at[pl.ds(r0 + 128 * b, 128)])
        pltpu.sync_copy(acc_sh.at[pl.ds(r0 + 512, 120)], frow.at[pl.ds(0, 120)])
        pltpu.sync_copy(frow.at[pl.ds(0, 120)], acc_out.at[pl.ds(r0 + 512, 120)])

    @pl.when(c == 0)
    def _():
        _wout(accA)

    @pl.when(c == 1)
    def _():
        _wout(accB)


_agg_kernel = pl.kernel(
    _agg_body,
    out_type=(
        jax.ShapeDtypeStruct((NP_, 128), jnp.float32),
        jax.ShapeDtypeStruct((NP_, 128), jnp.float32),
    ),
    mesh=_mesh,
    compiler_params=_cp,
    scratch_types=[
        pltpu.VMEM_SHARED((NP_, 128), jnp.float32),
        pltpu.VMEM((8, KCH), jnp.int32),
        pltpu.VMEM((8, KCH), jnp.int32),
        pltpu.VMEM((16, 128), jnp.float32),
        pltpu.VMEM((16, 128), jnp.float32),
        pltpu.VMEM((KCH, 128), jnp.float32),
        pltpu.SemaphoreType.DMA,
        pltpu.SemaphoreType.DMA,
        pltpu.SemaphoreType.DMA,
    ],
)


def _den_body(dstm, exoA, exoB,
              denA, denB,
              den_sh, dstv, exbuf0, exbuf1, exwide, sems, seme0, seme1):
    c = lax.axis_index("c")
    s = lax.axis_index("s")

    zv = jnp.zeros((16,), jnp.float32)

    @pl.loop(0, KCH)
    def _(k):
        for j in range(8):
            exwide[k, pl.ds(16 * j, 16)] = zv

    r0 = s * ROWS_PT
    for b in range(4):
        pltpu.sync_copy(exwide, den_sh.at[pl.ds(r0 + 128 * b, 128)])
    pltpu.sync_copy(exwide.at[pl.ds(0, 120)], den_sh.at[pl.ds(r0 + 512, 120)])

    plsc.subcore_barrier()

    @pl.loop(0, NB)
    def _(b):
        pltpu.sync_copy(dstm.at[pl.ds(s * 80 + 8 * b, 8)], dstv)
        for r in range(8):
            m16 = (s * 80 + 8 * b + r) * 16
            exb, seme = (exbuf0, seme0) if r % 2 == 0 else (exbuf1, seme1)
            nexb, nseme = (exbuf1, seme1) if r % 2 == 0 else (exbuf0, seme0)

            if r == 0:
                @pl.when(c == 0)
                def _():
                    pltpu.sync_copy(exoA.at[pl.ds(m16, 16)], exb)

                @pl.when(c == 1)
                def _():
                    pltpu.sync_copy(exoB.at[pl.ds(m16, 16)], exb)
            else:
                pltpu.make_async_copy(exwide, den_sh.at[dstv.at[r - 1]], sems).wait()

            if r < 7:
                @pl.when(c == 0)
                def _():
                    pltpu.async_copy(exoA.at[pl.ds(m16 + 16, 16)], nexb, nseme)

                @pl.when(c == 1)
                def _():
                    pltpu.async_copy(exoB.at[pl.ds(m16 + 16, 16)], nexb, nseme)

            if r > 0:
                pltpu.make_async_copy(exoA.at[pl.ds(m16, 16)], exb, seme).wait()

            @plsc.parallel_loop(0, KCH, unroll=4)
            def _(k):
                pair = exb[k >> 3, pl.ds((k & 7) * 16, 16)]
                v0 = pair * jnp.float32(0) + pair[0]
                v1 = pair * jnp.float32(0) + pair[1]
                for j in range(4):
                    exwide[k, pl.ds(16 * j, 16)] = v0
                for j in range(4, 8):
                    exwide[k, pl.ds(16 * j, 16)] = v1

            pltpu.async_copy(exwide, den_sh.at[dstv.at[r]], sems, add=True)

        pltpu.make_async_copy(exwide, den_sh.at[dstv.at[7]], sems).wait()

    plsc.subcore_barrier()

    def _wout(den_out):
        for b in range(4):
            pltpu.sync_copy(den_sh.at[pl.ds(r0 + 128 * b, 128)], exwide)
            pltpu.sync_copy(exwide, den_out.at[pl.ds(r0 + 128 * b, 128)])
        pltpu.sync_copy(den_sh.at[pl.ds(r0 + 512, 120)], exwide.at[pl.ds(0, 120)])
        pltpu.sync_copy(exwide.at[pl.ds(0, 120)], den_out.at[pl.ds(r0 + 512, 120)])

    @pl.when(c == 0)
    def _():
        _wout(denA)

    @pl.when(c == 1)
    def _():
        _wout(denB)


_den_kernel = pl.kernel(
    _den_body,
    out_type=(
        jax.ShapeDtypeStruct((NP_, 128), jnp.float32),
        jax.ShapeDtypeStruct((NP_, 128), jnp.float32),
    ),
    mesh=_mesh,
    compiler_params=_cp,
    scratch_types=[
        pltpu.VMEM_SHARED((NP_, 128), jnp.float32),
        pltpu.VMEM((8, KCH), jnp.int32),
        pltpu.VMEM((16, 128), jnp.float32),
        pltpu.VMEM((16, 128), jnp.float32),
        pltpu.VMEM((KCH, 128), jnp.float32),
        pltpu.SemaphoreType.DMA,
        pltpu.SemaphoreType.DMA,
        pltpu.SemaphoreType.DMA,
    ],
)


def _feat_body(x_ref, W_ref, twA_ref, twB_ref, fA_ref, fB_ref, tA_ref, tB_ref):
    f = jnp.dot(x_ref[...], W_ref[...], preferred_element_type=jnp.float32)
    fA_ref[...] = f[:, :128]
    fB_ref[...] = f[:, 128:]
    tA_ref[...] = jnp.dot(f, twA_ref[...], preferred_element_type=jnp.float32)
    tB_ref[...] = jnp.dot(f, twB_ref[...], preferred_element_type=jnp.float32)


_BN = 1264

_feat_kernel = pl.pallas_call(
    _feat_body,
    grid=(NP_ // _BN,),
    in_specs=[
        pl.BlockSpec((_BN, IN), lambda i: (i, 0)),
        pl.BlockSpec((IN, HD), lambda i: (0, 0)),
        pl.BlockSpec((HD, 8), lambda i: (0, 0)),
        pl.BlockSpec((HD, 8), lambda i: (0, 0)),
    ],
    out_specs=[
        pl.BlockSpec((_BN, 128), lambda i: (i, 0)),
        pl.BlockSpec((_BN, 128), lambda i: (i, 0)),
        pl.BlockSpec((_BN, 8), lambda i: (i, 0)),
        pl.BlockSpec((_BN, 8), lambda i: (i, 0)),
    ],
    out_shape=(
        jax.ShapeDtypeStruct((NP_, 128), jnp.float32),
        jax.ShapeDtypeStruct((NP_, 128), jnp.float32),
        jax.ShapeDtypeStruct((NP_, 8), jnp.float32),
        jax.ShapeDtypeStruct((NP_, 8), jnp.float32),
    ),
)


def _gat(x, edge_index, W, al, ar, bias):
    def tw_for(h0):
        t = jnp.zeros((HD, 8), jnp.float32)
        for j, (vec, h) in enumerate([(al, h0), (al, h0 + 1), (ar, h0), (ar, h0 + 1)]):
            t = t.at[h * D:(h + 1) * D, j].set(vec[h])
        return t

    x_pad = jnp.zeros((NP_, IN), jnp.float32).at[:N].set(x)
    featA, featB, tabA8, tabB8 = _feat_kernel(x_pad, W, tw_for(0), tw_for(2))
    tabA = tabA8[:, :4].reshape(-1)
    tabB = tabB8[:, :4].reshape(-1)
    srcm = jnp.full((EP,), N, jnp.int32).at[:E].set(edge_index[0]).reshape(NCH, KCH)
    dstm = jnp.full((EP,), N, jnp.int32).at[:E].set(edge_index[1]).reshape(NCH, KCH)

    exoA, exoB = _ex_kernel(tabA, tabB, srcm, dstm)
    accA, accB = _agg_kernel(featA, featB, srcm, dstm, exoA, exoB)
    denA, denB = _den_kernel(dstm, exoA, exoB)

    acc = jnp.concatenate([accA[:N], accB[:N]], axis=1).reshape(N, H, D)
    den4 = jnp.concatenate(
        [denA[:N, 0:1], denA[:N, 64:65], denB[:N, 0:1], denB[:N, 64:65]], axis=1)
    den4 = jnp.maximum(den4, 1e-16)
    rst = acc / den4[:, :, None] + bias.reshape(1, H, D)
    return jax.nn.elu(rst)


def _ovw_body(scidx, rows_in, o0, o1, o2,
              acc_sh, idxv, frow):
    c = lax.axis_index("c")
    s = lax.axis_index("s")
    zv = jnp.zeros((16,), jnp.float32)
    r0 = s * ROWS_PT
    outs = [o0, o1, o2]
    for p in range(3):
        @pl.loop(0, KCH)
        def _(k):
            for j in range(8):
                frow[k, pl.ds(16 * j, 16)] = zv

        for b in range(4):
            pltpu.sync_copy(frow, acc_sh.at[pl.ds(r0 + 128 * b, 128)])
        pltpu.sync_copy(frow.at[pl.ds(0, 120)], acc_sh.at[pl.ds(r0 + 512, 120)])
        plsc.subcore_barrier()

        chunk = s
        @pl.when(c == 0)
        def _():
            pltpu.sync_copy(scidx.at[pl.ds(16 * p + 8 * (s // 8), 8)], idxv)
            pltpu.sync_copy(rows_in.at[pl.ds((16 * p + s) * 128, 128)], frow)
            pltpu.sync_copy(frow, acc_sh.at[idxv.at[s % 8]], add=True)

        plsc.subcore_barrier()

        def _wout(out):
            for b in range(4):
                pltpu.sync_copy(acc_sh.at[pl.ds(r0 + 128 * b, 128)], frow)
                pltpu.sync_copy(frow, out.at[pl.ds(r0 + 128 * b, 128)])
            pltpu.sync_copy(acc_sh.at[pl.ds(r0 + 512, 120)], frow.at[pl.ds(0, 120)])
            pltpu.sync_copy(frow.at[pl.ds(0, 120)], out.at[pl.ds(r0 + 512, 120)])

        @pl.when(c == 0)
        def _():
            _wout(outs[p])
        plsc.subcore_barrier()


_ovw_kernel = pl.kernel(
    _ovw_body,
    out_type=(
        jax.ShapeDtypeStruct((NP_, 128), jnp.float32),
        jax.ShapeDtypeStruct((NP_, 128), jnp.float32),
        jax.ShapeDtypeStruct((NP_, 128), jnp.float32),
    ),
    mesh=_mesh,
    compiler_params=_cp,
    scratch_types=[
        pltpu.VMEM_SHARED((NP_, 128), jnp.float32),
        pltpu.VMEM((8, KCH), jnp.int32),
        pltpu.VMEM((KCH, 128), jnp.float32),
    ],
)


def _sem_att(z, W1, b1, W2):
    h = jnp.tanh(z @ W1 + b1)
    w = (h @ W2).mean(0)
    beta = jax.nn.softmax(w, axis=0)
    return (beta[None, :, :] * z).sum(1)


def _last_occ_idx(char):
    idx = jnp.arange(C)
    eq = char[None, :] == char[:, None]
    has_later = (eq & (idx[None, :] > idx[:, None])).any(axis=1)
    return jnp.where(has_later, N, char).astype(jnp.int32)


def kernel(x0, x1, x2, edge_index0, edge_index1, edge_index2, char0, char1, char2, W0, al0, ar0, bias0, W1, al1, ar1, bias1, W2, al2, ar2, bias2, sW1, sb1, sW2, aW1, ab1, aW2):
    h1 = _gat(x0, edge_index0, W0, al0, ar0, bias0)
    hh = _gat(x2, edge_index2, W2, al2, ar2, bias2)
    h2 = _gat(x1, edge_index1, W1, al1, ar1, bias1)

    se = jnp.stack([h1[char0].reshape(C, HD), hh[char2].reshape(C, HD)], axis=1)
    s = _sem_att(se, sW1, sb1, sW2)
    se2 = jnp.stack([s, h2[char1].reshape(C, HD)], axis=1)
    sem = _sem_att(se2, aW1, ab1, aW2)
    semr = sem.reshape(C, H, D)
    semm = semr.mean(axis=1)

    scidx = jnp.stack([_last_occ_idx(char0), _last_occ_idx(char1),
                       _last_occ_idx(char2)]).reshape(48, 128)
    rows_in = jnp.concatenate(
        [jnp.concatenate([semm, jnp.ones((C, 128 - D), jnp.float32)], axis=1)] * 3,
        axis=0)
    d0, d1, d2 = _ovw_kernel(scidx, rows_in)

    def _finish(h_nhd, dbuf):
        sel = dbuf[:N, D:D + 1] > 0
        return jnp.where(sel, dbuf[:N, :D], h_nhd.mean(axis=1))

    return (semr, _finish(h1, d0), _finish(h2, d1), _finish(hh, d2))
